# baseline (speedup 1.0000x reference)
import functools

import jax
import jax.numpy as jnp
from jax.experimental import pallas as pl

F32 = jnp.float32
BF16 = jnp.bfloat16
PREC = jax.lax.Precision.HIGHEST


def _dot(a, b):
    return jnp.dot(a, b, precision=PREC, preferred_element_type=F32)


def _bdot(a, b):
    return jnp.dot(a.astype(BF16), b, preferred_element_type=F32)


def _prep(x):
    n, h, w, c = x.shape
    xp = jnp.pad(x, ((0, 0), (1, 1), (1, 1), (0, 0)))
    hp, wp = h + 2, w + 2
    flat = xp.reshape(n, hp * wp, c)
    need = hp * wp + 2
    need += (-need) % 8
    return jnp.pad(flat, ((0, 0), (0, need - hp * wp), (0, 0)))


def _pick_tiles(hout, wp):
    lout = hout * wp
    if lout <= 1024:
        return lout, 1
    if hout % 8 == 0:
        return 8 * wp, hout // 8
    return 4 * wp, hout // 4


def _conv_body(*refs, nx, ng, offs, acts, tl, chunk):
    base = pl.program_id(1) * tl
    xrefs = refs[:nx]
    for g in range(ng):
        wref = refs[nx + 2 * g]
        bref = refs[nx + 2 * g + 1]
        oref = refs[nx + 2 * ng + g]
        cout = wref.shape[1]
        acc = None
        cur, cur_k = [], 0
        tap_sizes = [xrefs[xi].shape[2] for xi, off in offs[g]]
        wof = 0
        chunks = []
        for (xi, off), ksz in zip(offs[g], tap_sizes):
            if cur_k + ksz > chunk and cur:
                chunks.append((cur, cur_k))
                cur, cur_k = [], 0
            cur.append((xi, off, wof))
            cur_k += ksz
            wof += ksz
        if cur:
            chunks.append((cur, cur_k))
        for taps, ksz in chunks:
            pieces = [xrefs[xi][0, pl.ds(base + off, tl), :].astype(BF16)
                      for xi, off, _ in taps]
            sl = pieces[0] if len(pieces) == 1 else jnp.concatenate(pieces, 1)
            w0 = taps[0][2]
            part = jnp.dot(sl, wref[pl.ds(w0, ksz), :],
                           preferred_element_type=F32)
            acc = part if acc is None else acc + part
        acc = acc + bref[...]
        if acts[g]:
            acc = jnp.maximum(acc, 0.0)
        oref[0] = acc


def _run_conv(xs, groups, hout, wp):
    n = xs[0].shape[0]
    tl, nt = _pick_tiles(hout, wp)
    lout = hout * wp
    body = functools.partial(
        _conv_body,
        nx=len(xs),
        ng=len(groups),
        offs=tuple(tuple(g[2]) for g in groups),
        acts=tuple(bool(g[3]) for g in groups),
        tl=tl,
        chunk=256,
    )
    in_specs = [
        pl.BlockSpec((1, x.shape[1], x.shape[2]), lambda nn, tt: (nn, 0, 0))
        for x in xs
    ]
    operands = list(xs)
    out_specs, out_shapes = [], []
    for w, b, _, _ in groups:
        cout = w.shape[1]
        in_specs.append(pl.BlockSpec(w.shape, lambda nn, tt: (0, 0)))
        in_specs.append(pl.BlockSpec((1, cout), lambda nn, tt: (0, 0)))
        operands.append(w.astype(BF16))
        operands.append(b.reshape(1, cout).astype(F32))
        out_specs.append(pl.BlockSpec((1, tl, cout), lambda nn, tt: (nn, tt, 0)))
        out_shapes.append(jax.ShapeDtypeStruct((n, nt * tl, cout), F32))
    outs = pl.pallas_call(
        body,
        grid=(n, nt),
        in_specs=in_specs,
        out_specs=out_specs,
        out_shape=out_shapes,
    )(*operands)
    res = []
    for o, (w, _, _, _) in zip(outs, groups):
        cout = w.shape[1]
        res.append(o[:, :lout].reshape(n, hout, wp, cout)[:, :, : wp - 2])
    return res


def _conv3(x, w, b, relu):
    n, h, wd, cin = x.shape
    wp = wd + 2
    xp = _prep(x)
    offs = [(0, dh * wp + dw) for dh in range(3) for dw in range(3)]
    wt = jnp.concatenate(
        [w[:, :, dh, dw].T for dh in range(3) for dw in range(3)], 0)
    return _run_conv([xp], [(wt, b, offs, relu)], h, wp)[0]


def _conv3_cat(xa, xb, w, b, relu):
    n, h, wd, ca = xa.shape
    wp = wd + 2
    xpa, xpb = _prep(xa), _prep(xb)
    offs, wts = [], []
    for dh in range(3):
        for dw in range(3):
            offs.append((0, dh * wp + dw))
            wts.append(w[:, :ca, dh, dw].T)
            offs.append((1, dh * wp + dw))
            wts.append(w[:, ca:, dh, dw].T)
    wt = jnp.concatenate(wts, 0)
    return _run_conv([xpa, xpb], [(wt, b, offs, relu)], h, wp)[0]


def _s2conv(x, w, b, relu):
    n, h, wd, cin = x.shape
    hq = h // 2
    wp = hq + 2
    quarters = [_prep(x[:, a::2, bb::2, :]) for a in (0, 1) for bb in (0, 1)]
    offs, wts = [], []
    for kh in range(4):
        a, dh = (kh + 1) % 2, (kh + 1) // 2
        for kw in range(4):
            bb, dw = (kw + 1) % 2, (kw + 1) // 2
            offs.append((a * 2 + bb, dh * wp + dw))
            wts.append(w[:, :, kh, kw].T)
    wt = jnp.concatenate(wts, 0)
    return _run_conv(quarters, [(wt, b, offs, relu)], hq, wp)[0]


def _convt(x, w, b, relu):
    n, h, wd, cin = x.shape
    wp = wd + 2
    xp = _prep(x)
    cout = w.shape[1]
    groups = []
    for a in (0, 1):
        htaps = [(0, 3), (1, 1)] if a == 0 else [(1, 2), (2, 0)]
        for bb in (0, 1):
            wtaps = [(0, 3), (1, 1)] if bb == 0 else [(1, 2), (2, 0)]
            offs, wts = [], []
            for dh, kh in htaps:
                for dw, kw in wtaps:
                    offs.append((0, dh * wp + dw))
                    wts.append(w[:, :, kh, kw])
            groups.append((jnp.concatenate(wts, 0), b, offs, relu))
    outs = _run_conv([xp], groups, h, wp)
    z = jnp.stack(outs).reshape(2, 2, n, h, wd, cout)
    z = jnp.transpose(z, (2, 3, 0, 4, 1, 5)).reshape(n, 2 * h, 2 * wd, cout)
    return z


def _resblock_body(x_ref, w1_ref, b1_ref, w2_ref, b2_ref, o_ref, *, offs, tl,
                   wp, relu_out):
    base = pl.program_id(1) * tl
    c = x_ref.shape[2]
    ntap_chunk = max(1, 256 // c)
    acc = None
    for s in range(0, len(offs), ntap_chunk):
        grp = offs[s:s + ntap_chunk]
        pieces = [jnp.maximum(x_ref[0, pl.ds(base + off, tl), :], 0.0)
                  .astype(BF16) for off in grp]
        sl = pieces[0] if len(pieces) == 1 else jnp.concatenate(pieces, 1)
        part = jnp.dot(sl, w1_ref[s * c:(s + len(grp)) * c, :],
                       preferred_element_type=F32)
        acc = part if acc is None else acc + part
    hmid = jnp.maximum(acc + b1_ref[...], 0.0)
    h2 = _bdot(hmid, w2_ref[...]) + b2_ref[...]
    res = x_ref[0, pl.ds(base + wp + 1, tl), :] + h2
    if relu_out:
        res = jnp.maximum(res, 0.0)
    o_ref[0] = res


def _resblock(x, w1, b1, w2, b2, relu_out):
    n, h, wd, c = x.shape
    wp = wd + 2
    xp = _prep(x)
    tl, nt = _pick_tiles(h, wp)
    offs = tuple(dh * wp + dw for dh in range(3) for dw in range(3))
    w1t = jnp.concatenate(
        [w1[:, :, dh, dw].T for dh in range(3) for dw in range(3)], 0)
    w2t = w2[:, :, 0, 0].T
    c1 = w1.shape[0]
    body = functools.partial(_resblock_body, offs=offs, tl=tl, wp=wp,
                             relu_out=relu_out)
    out = pl.pallas_call(
        body,
        grid=(n, nt),
        in_specs=[
            pl.BlockSpec((1, xp.shape[1], c), lambda nn, tt: (nn, 0, 0)),
            pl.BlockSpec(w1t.shape, lambda nn, tt: (0, 0)),
            pl.BlockSpec((1, c1), lambda nn, tt: (0, 0)),
            pl.BlockSpec(w2t.shape, lambda nn, tt: (0, 0)),
            pl.BlockSpec((1, c), lambda nn, tt: (0, 0)),
        ],
        out_specs=pl.BlockSpec((1, tl, c), lambda nn, tt: (nn, tt, 0)),
        out_shape=jax.ShapeDtypeStruct((n, nt * tl, c), F32),
    )(xp, w1t.astype(BF16), b1.reshape(1, c1).astype(F32), w2t.astype(BF16),
      b2.reshape(1, c).astype(F32))
    return out[:, : h * wp].reshape(n, h, wp, c)[:, :, : wp - 2]


def _vq_body(x_ref, qw_ref, qb_ref, cb_ref, cbt_ref, cbsq_ref, oq_ref, oi_ref,
             oc_ref, *, tb, ncode):
    t = pl.program_id(0)
    x = x_ref[...]
    z = _bdot(x, qw_ref[...]) + qb_ref[...]
    s = _bdot(z, cbt_ref[...])
    d = cbsq_ref[...] - 2.0 * s
    m = jnp.min(d, axis=1, keepdims=True)
    lane = jax.lax.broadcasted_iota(jnp.int32, (tb, ncode), 1)
    idx = jnp.min(jnp.where(d <= m, lane, ncode), axis=1)
    oh = (lane == idx[:, None]).astype(F32)
    q = _dot(oh, cb_ref[...])
    oq_ref[...] = q
    oi_ref[...] = idx[:, None]
    part = jnp.sum((q - z) * (q - z))

    @pl.when(t == 0)
    def _():
        oc_ref[...] = jnp.zeros_like(oc_ref)

    oc_ref[...] = oc_ref[...] + part


def _vq(tokens, qw, qb, cb, tb):
    ntok, cin = tokens.shape
    ncode, dd = cb.shape
    nt = ntok // tb
    qwt = qw[:, :, 0, 0].T.astype(BF16)
    cbt = cb.T.astype(BF16)
    cbsq = jnp.sum(cb.astype(F32) * cb.astype(F32), -1)[None, :]
    body = functools.partial(_vq_body, tb=tb, ncode=ncode)
    q, idx, csum = pl.pallas_call(
        body,
        grid=(nt,),
        in_specs=[
            pl.BlockSpec((tb, cin), lambda tt: (tt, 0)),
            pl.BlockSpec((cin, dd), lambda tt: (0, 0)),
            pl.BlockSpec((1, dd), lambda tt: (0, 0)),
            pl.BlockSpec((ncode, dd), lambda tt: (0, 0)),
            pl.BlockSpec((dd, ncode), lambda tt: (0, 0)),
            pl.BlockSpec((1, ncode), lambda tt: (0, 0)),
        ],
        out_specs=[
            pl.BlockSpec((tb, dd), lambda tt: (tt, 0)),
            pl.BlockSpec((tb, 1), lambda tt: (tt, 0)),
            pl.BlockSpec((1, 1), lambda tt: (0, 0)),
        ],
        out_shape=[
            jax.ShapeDtypeStruct((ntok, dd), F32),
            jax.ShapeDtypeStruct((ntok, 1), jnp.int32),
            jax.ShapeDtypeStruct((1, 1), F32),
        ],
    )(tokens.astype(F32), qwt, qb.reshape(1, dd).astype(F32), cb.astype(F32),
      cbt, cbsq)
    return q, idx[:, 0], csum[0, 0]


def kernel(input, eb_w1, eb_b1, eb_w2, eb_b2, eb_w3, eb_b3,
           eb_r1_w1, eb_r1_b1, eb_r1_w2, eb_r1_b2,
           eb_r2_w1, eb_r2_b1, eb_r2_w2, eb_r2_b2,
           et_w1, et_b1, et_w2, et_b2,
           et_r1_w1, et_r1_b1, et_r1_w2, et_r1_b2,
           et_r2_w1, et_r2_b1, et_r2_w2, et_r2_b2,
           qct_w, qct_b, qcb_w, qcb_b, cb_t, cb_b, ut_w, ut_b,
           d_w1, d_b1,
           d_r1_w1, d_r1_b1, d_r1_w2, d_r1_b2,
           d_r2_w1, d_r2_b1, d_r2_w2, d_r2_b2,
           dct1_w, dct1_b, dct2_w, dct2_b):
    n = input.shape[0]
    x = jnp.transpose(input, (0, 2, 3, 1))

    h = _s2conv(x, eb_w1, eb_b1, relu=True)
    h = _s2conv(h, eb_w2, eb_b2, relu=True)
    h = _conv3(h, eb_w3, eb_b3, relu=False)
    h = _resblock(h, eb_r1_w1, eb_r1_b1, eb_r1_w2, eb_r1_b2, relu_out=False)
    enc_b = _resblock(h, eb_r2_w1, eb_r2_b1, eb_r2_w2, eb_r2_b2, relu_out=True)

    h = _s2conv(enc_b, et_w1, et_b1, relu=True)
    h = _conv3(h, et_w2, et_b2, relu=False)
    h = _resblock(h, et_r1_w1, et_r1_b1, et_r1_w2, et_r1_b2, relu_out=False)
    enc_t = _resblock(h, et_r2_w1, et_r2_b1, et_r2_w2, et_r2_b2, relu_out=True)

    ht, wt_ = enc_t.shape[1], enc_t.shape[2]
    hb, wb = enc_b.shape[1], enc_b.shape[2]
    tok_t = enc_t.reshape(n * ht * wt_, 128)
    tok_b = enc_b.reshape(n * hb * wb, 128)
    q_t, id_t, cs_t = _vq(tok_t, qct_w, qct_b, cb_t, tb=112)
    q_b, id_b, cs_b = _vq(tok_b, qcb_w, qcb_b, cb_b, tb=112)
    quant_t = q_t.reshape(n, ht, wt_, 64)
    quant_b = q_b.reshape(n, hb, wb, 64)

    up_t = _convt(quant_t, ut_w, ut_b, relu=False)
    h = _conv3_cat(up_t, quant_b, d_w1, d_b1, relu=False)
    h = _resblock(h, d_r1_w1, d_r1_b1, d_r1_w2, d_r1_b2, relu_out=False)
    h = _resblock(h, d_r2_w1, d_r2_b1, d_r2_w2, d_r2_b2, relu_out=True)
    h = _convt(h, dct1_w, dct1_b, relu=True)
    dec = _convt(h, dct2_w, dct2_b, relu=False)
    dec = jnp.transpose(dec, (0, 3, 1, 2))

    diff = (cs_t / (n * ht * wt_ * 64) + cs_b / (n * hb * wb * 64)).reshape(1)
    id_t = id_t.reshape(n, ht * wt_)
    id_b = id_b.reshape(n, hb * wb)
    return dec, diff, id_b, id_t
